# baseline (speedup 1.0000x reference)
import jax
import jax.numpy as jnp
from jax import lax
from jax.experimental import pallas as pl
from jax.experimental.pallas import tpu as pltpu
from jax.experimental.pallas import tpu_sc as plsc

N = 10000
NP = 10240
E = 320000
D = 128
NC = 2
NS = 16
L = 16
NW = NC * NS
EPW = E // NW
CH = 80
NCH = EPW // CH
RPA = NP // NS
RB = 1024

_MESH = dict(core_axis_name="c", subcore_axis_name="s")


def _deg_body(dst_hbm, ones_hbm, zeros_hbm, out_hbm, buf, dst_v, hist_sh):
    c = lax.axis_index("c")
    s = lax.axis_index("s")
    wid = s * NC + c

    pltpu.sync_copy(dst_hbm.at[wid], dst_v)
    pltpu.sync_copy(zeros_hbm, buf)
    for q in range(RPA // CH):
        pltpu.sync_copy(buf, hist_sh.at[pl.ds(s * RPA + q * CH, CH)])
    pltpu.sync_copy(ones_hbm, buf)
    plsc.subcore_barrier()

    def step(j, carry):
        pltpu.sync_copy(buf, hist_sh.at[dst_v.at[j]], add=True)
        return carry
    lax.fori_loop(0, NCH, step, 0)
    plsc.subcore_barrier()
    pltpu.sync_copy(hist_sh.at[pl.ds(s * RPA, RPA)],
                    out_hbm.at[pl.ds(c * NP + s * RPA, RPA)])


def _degrees(dst3, ones2d, zeros2d):
    f = pl.kernel(
        _deg_body,
        out_type=jax.ShapeDtypeStruct((NC * NP, D), jnp.float32),
        mesh=plsc.VectorSubcoreMesh(**_MESH),
        scratch_types=[
            pltpu.VMEM((CH, D), jnp.float32),
            pltpu.VMEM((NCH, CH), jnp.int32),
            pltpu.VMEM_SHARED((NP, D), jnp.float32),
        ],
    )
    return f(dst3, ones2d, zeros2d)


RBB = 1000


def _norm_body(hists_ref, feat_ref, h_ref, norm_ref):
    i = pl.program_id(0)
    blk = hists_ref[...]
    deg = blk[0, :, 0] + blk[1, :, 0]
    nrm = lax.rsqrt(jnp.maximum(deg, 1.0))
    h_ref[...] = feat_ref[...] * nrm[:, None]
    norm_ref[...] = nrm[:, None]


def _prenorm(hists, feat):
    return pl.pallas_call(
        _norm_body,
        grid=(N // RBB,),
        in_specs=[
            pl.BlockSpec((NC, RBB, D), lambda i: (0, i, 0)),
            pl.BlockSpec((RBB, D), lambda i: (i, 0)),
        ],
        out_specs=[
            pl.BlockSpec((RBB, D), lambda i: (i, 0)),
            pl.BlockSpec((RBB, 1), lambda i: (i, 0)),
        ],
        out_shape=[
            jax.ShapeDtypeStruct((N, D), jnp.float32),
            jax.ShapeDtypeStruct((N, 1), jnp.float32),
        ],
    )(hists, feat)


def _agg_body(h_hbm, src_hbm, dst_hbm, zeros_hbm, out_hbm, src_v, iv0, iv1,
              buf0, buf1, agg_sh, semA0, semA1, semG0, semG1, semD0, semD1):
    c = lax.axis_index("c")
    s = lax.axis_index("s")
    wid = s * NC + c
    base = wid * EPW

    def gather(j, buf, sem):
        return pltpu.make_async_copy(
            h_hbm.at[src_v.at[pl.ds(j * CH, CH)]], buf, sem)

    def start_iv(j, iv, sem):
        pltpu.make_async_copy(dst_hbm.at[pl.ds(base + j * CH, CH)], iv,
                              sem).start()

    def wait_iv(j, iv, sem):
        pltpu.make_async_copy(dst_hbm.at[pl.ds(base + j * CH, CH)], iv,
                              sem).wait()

    pltpu.sync_copy(zeros_hbm, buf0)
    for q in range(RPA // CH):
        pltpu.sync_copy(buf0, agg_sh.at[pl.ds(s * RPA + q * CH, CH)])
    pltpu.sync_copy(src_hbm.at[pl.ds(base, EPW)], src_v)
    plsc.subcore_barrier()

    def add_start(buf, iv, sem):
        pltpu.async_copy(buf, agg_sh.at[iv], sem, add=True)

    def add_wait(buf, iv, sem):
        pltpu.make_async_copy(buf, agg_sh.at[iv], sem).wait()

    start_iv(0, iv0, semA0)
    gather(0, buf0, semG0).start()

    def step(g, carry):
        j0 = 2 * g
        start_iv(j0 + 1, iv1, semA1)
        gather(j0 + 1, buf1, semG1).start()
        gather(j0, buf0, semG0).wait()
        wait_iv(j0, iv0, semA0)
        add_start(buf0, iv0, semD0)
        gather(j0 + 1, buf1, semG1).wait()
        wait_iv(j0 + 1, iv1, semA1)
        add_start(buf1, iv1, semD1)
        add_wait(buf0, iv0, semD0)
        start_iv(j0 + 2, iv0, semA0)
        gather(j0 + 2, buf0, semG0).start()
        add_wait(buf1, iv1, semD1)
        return carry
    lax.fori_loop(0, (NCH - 1) // 2, step, 0)
    gather(NCH - 1, buf0, semG0).wait()
    wait_iv(NCH - 1, iv0, semA0)
    pltpu.sync_copy(buf0, agg_sh.at[iv0], add=True)
    plsc.subcore_barrier()
    pltpu.sync_copy(agg_sh.at[pl.ds(s * RPA, RPA)],
                    out_hbm.at[pl.ds(c * NP + s * RPA, RPA)])


def _aggregate(h, src_flat, dst_flat, zeros2d):
    f = pl.kernel(
        _agg_body,
        out_type=jax.ShapeDtypeStruct((NC * NP, D), jnp.float32),
        mesh=plsc.VectorSubcoreMesh(**_MESH),
        scratch_types=[
            pltpu.VMEM((EPW,), jnp.int32),
            pltpu.VMEM((CH,), jnp.int32),
            pltpu.VMEM((CH,), jnp.int32),
            pltpu.VMEM((CH, D), jnp.float32),
            pltpu.VMEM((CH, D), jnp.float32),
            pltpu.VMEM_SHARED((NP, D), jnp.float32),
            pltpu.SemaphoreType.DMA,
            pltpu.SemaphoreType.DMA,
            pltpu.SemaphoreType.DMA,
            pltpu.SemaphoreType.DMA,
            pltpu.SemaphoreType.DMA,
            pltpu.SemaphoreType.DMA,
        ],
    )
    return f(h, src_flat, dst_flat, zeros2d)


def _out_body(paggs_ref, norm_ref, w_ref, b_ref, out_ref):
    i = pl.program_id(0)
    blk = paggs_ref[...]
    agg = blk[0] + blk[1]
    hh = agg * norm_ref[...]
    out_ref[...] = lax.dot_general(
        hh, w_ref[...], (((1,), (1,)), ((), ())),
        preferred_element_type=jnp.float32) + b_ref[...]


def _project(paggs, norm, W, b2):
    return pl.pallas_call(
        _out_body,
        grid=(N // RBB,),
        in_specs=[
            pl.BlockSpec((NC, RBB, D), lambda i: (0, i, 0)),
            pl.BlockSpec((RBB, 1), lambda i: (i, 0)),
            pl.BlockSpec((D, D), lambda i: (0, 0)),
            pl.BlockSpec((1, D), lambda i: (0, 0)),
        ],
        out_specs=pl.BlockSpec((RBB, D), lambda i: (i, 0)),
        out_shape=jax.ShapeDtypeStruct((N, D), jnp.float32),
    )(paggs, norm, W, b2)


def kernel(feat, edge_index, W, b):
    ei = edge_index.astype(jnp.int32)
    src_flat = ei[0]
    dst_flat = ei[1]
    ones2d = jnp.ones((CH, D), jnp.float32)
    zeros2d = jnp.zeros((CH, D), jnp.float32)
    hists = _degrees(dst_flat.reshape(NW, NCH, CH), ones2d, zeros2d)
    h, norm = _prenorm(hists.reshape(NC, NP, D), feat)
    paggs = _aggregate(h, src_flat, dst_flat, zeros2d)
    return _project(paggs.reshape(NC, NP, D), norm, W, b.reshape(1, D))


# speedup vs baseline: 1.0020x; 1.0020x over previous
import jax
import jax.numpy as jnp
from jax import lax
from jax.experimental import pallas as pl
from jax.experimental.pallas import tpu as pltpu
from jax.experimental.pallas import tpu_sc as plsc

N = 10000
NP = 10240
E = 320000
D = 128
NC = 2
NS = 16
L = 16
NW = NC * NS
EPW = E // NW
CH = 80
NCH = EPW // CH
RPA = NP // NS
RB = 1024

_MESH = dict(core_axis_name="c", subcore_axis_name="s")


def _deg_body(dst_hbm, ones_hbm, zeros_hbm, out_hbm, buf, iv0, iv1, hist_sh,
              semA0, semA1):
    c = lax.axis_index("c")
    s = lax.axis_index("s")
    wid = s * NC + c
    base = wid * EPW

    def start_iv(j, iv, sem):
        pltpu.make_async_copy(dst_hbm.at[pl.ds(base + j * CH, CH)], iv,
                              sem).start()

    def wait_iv(j, iv, sem):
        pltpu.make_async_copy(dst_hbm.at[pl.ds(base + j * CH, CH)], iv,
                              sem).wait()

    pltpu.sync_copy(zeros_hbm, buf)
    for q in range(RPA // CH):
        pltpu.sync_copy(buf, hist_sh.at[pl.ds(s * RPA + q * CH, CH)])
    pltpu.sync_copy(ones_hbm, buf)
    plsc.subcore_barrier()

    start_iv(0, iv0, semA0)

    def step(g, carry):
        j0 = 2 * g
        start_iv(j0 + 1, iv1, semA1)
        wait_iv(j0, iv0, semA0)
        pltpu.sync_copy(buf, hist_sh.at[iv0], add=True)
        start_iv(j0 + 2, iv0, semA0)
        wait_iv(j0 + 1, iv1, semA1)
        pltpu.sync_copy(buf, hist_sh.at[iv1], add=True)
        return carry
    lax.fori_loop(0, (NCH - 1) // 2, step, 0)
    wait_iv(NCH - 1, iv0, semA0)
    pltpu.sync_copy(buf, hist_sh.at[iv0], add=True)
    plsc.subcore_barrier()
    pltpu.sync_copy(hist_sh.at[pl.ds(s * RPA, RPA)],
                    out_hbm.at[pl.ds(c * NP + s * RPA, RPA)])


def _degrees(dst_flat, ones2d, zeros2d):
    f = pl.kernel(
        _deg_body,
        out_type=jax.ShapeDtypeStruct((NC * NP, D), jnp.float32),
        mesh=plsc.VectorSubcoreMesh(**_MESH),
        scratch_types=[
            pltpu.VMEM((CH, D), jnp.float32),
            pltpu.VMEM((CH,), jnp.int32),
            pltpu.VMEM((CH,), jnp.int32),
            pltpu.VMEM_SHARED((NP, D), jnp.float32),
            pltpu.SemaphoreType.DMA,
            pltpu.SemaphoreType.DMA,
        ],
    )
    return f(dst_flat, ones2d, zeros2d)


RBB = 1000


def _norm_body(hists_ref, feat_ref, h_ref, norm_ref):
    i = pl.program_id(0)
    blk = hists_ref[...]
    deg = blk[0, :, 0] + blk[1, :, 0]
    nrm = lax.rsqrt(jnp.maximum(deg, 1.0))
    h_ref[...] = feat_ref[...] * nrm[:, None]
    norm_ref[...] = nrm[:, None]


def _prenorm(hists, feat):
    return pl.pallas_call(
        _norm_body,
        grid=(N // RBB,),
        in_specs=[
            pl.BlockSpec((NC, RBB, D), lambda i: (0, i, 0)),
            pl.BlockSpec((RBB, D), lambda i: (i, 0)),
        ],
        out_specs=[
            pl.BlockSpec((RBB, D), lambda i: (i, 0)),
            pl.BlockSpec((RBB, 1), lambda i: (i, 0)),
        ],
        out_shape=[
            jax.ShapeDtypeStruct((N, D), jnp.float32),
            jax.ShapeDtypeStruct((N, 1), jnp.float32),
        ],
    )(hists, feat)


def _agg_body(h_hbm, src_hbm, dst_hbm, zeros_hbm, out_hbm, src_v, iv0, iv1,
              buf0, buf1, agg_sh, semA0, semA1, semG0, semG1, semD0, semD1):
    c = lax.axis_index("c")
    s = lax.axis_index("s")
    wid = s * NC + c
    base = wid * EPW

    def gather(j, buf, sem):
        return pltpu.make_async_copy(
            h_hbm.at[src_v.at[pl.ds(j * CH, CH)]], buf, sem)

    def start_iv(j, iv, sem):
        pltpu.make_async_copy(dst_hbm.at[pl.ds(base + j * CH, CH)], iv,
                              sem).start()

    def wait_iv(j, iv, sem):
        pltpu.make_async_copy(dst_hbm.at[pl.ds(base + j * CH, CH)], iv,
                              sem).wait()

    pltpu.sync_copy(zeros_hbm, buf0)
    for q in range(RPA // CH):
        pltpu.sync_copy(buf0, agg_sh.at[pl.ds(s * RPA + q * CH, CH)])
    pltpu.sync_copy(src_hbm.at[pl.ds(base, EPW)], src_v)
    plsc.subcore_barrier()

    def add_start(buf, iv, sem):
        pltpu.async_copy(buf, agg_sh.at[iv], sem, add=True)

    def add_wait(buf, iv, sem):
        pltpu.make_async_copy(buf, agg_sh.at[iv], sem).wait()

    start_iv(0, iv0, semA0)
    gather(0, buf0, semG0).start()

    def step(g, carry):
        j0 = 2 * g
        start_iv(j0 + 1, iv1, semA1)
        gather(j0 + 1, buf1, semG1).start()
        gather(j0, buf0, semG0).wait()
        wait_iv(j0, iv0, semA0)
        add_start(buf0, iv0, semD0)
        gather(j0 + 1, buf1, semG1).wait()
        wait_iv(j0 + 1, iv1, semA1)
        add_start(buf1, iv1, semD1)
        add_wait(buf0, iv0, semD0)
        start_iv(j0 + 2, iv0, semA0)
        gather(j0 + 2, buf0, semG0).start()
        add_wait(buf1, iv1, semD1)
        return carry
    lax.fori_loop(0, (NCH - 1) // 2, step, 0)
    gather(NCH - 1, buf0, semG0).wait()
    wait_iv(NCH - 1, iv0, semA0)
    pltpu.sync_copy(buf0, agg_sh.at[iv0], add=True)
    plsc.subcore_barrier()
    pltpu.sync_copy(agg_sh.at[pl.ds(s * RPA, RPA)],
                    out_hbm.at[pl.ds(c * NP + s * RPA, RPA)])


def _aggregate(h, src_flat, dst_flat, zeros2d):
    f = pl.kernel(
        _agg_body,
        out_type=jax.ShapeDtypeStruct((NC * NP, D), jnp.float32),
        mesh=plsc.VectorSubcoreMesh(**_MESH),
        scratch_types=[
            pltpu.VMEM((EPW,), jnp.int32),
            pltpu.VMEM((CH,), jnp.int32),
            pltpu.VMEM((CH,), jnp.int32),
            pltpu.VMEM((CH, D), jnp.float32),
            pltpu.VMEM((CH, D), jnp.float32),
            pltpu.VMEM_SHARED((NP, D), jnp.float32),
            pltpu.SemaphoreType.DMA,
            pltpu.SemaphoreType.DMA,
            pltpu.SemaphoreType.DMA,
            pltpu.SemaphoreType.DMA,
            pltpu.SemaphoreType.DMA,
            pltpu.SemaphoreType.DMA,
        ],
    )
    return f(h, src_flat, dst_flat, zeros2d)


def _out_body(paggs_ref, norm_ref, w_ref, b_ref, out_ref):
    i = pl.program_id(0)
    blk = paggs_ref[...]
    agg = blk[0] + blk[1]
    hh = agg * norm_ref[...]
    out_ref[...] = lax.dot_general(
        hh, w_ref[...], (((1,), (1,)), ((), ())),
        preferred_element_type=jnp.float32) + b_ref[...]


def _project(paggs, norm, W, b2):
    return pl.pallas_call(
        _out_body,
        grid=(N // RBB,),
        in_specs=[
            pl.BlockSpec((NC, RBB, D), lambda i: (0, i, 0)),
            pl.BlockSpec((RBB, 1), lambda i: (i, 0)),
            pl.BlockSpec((D, D), lambda i: (0, 0)),
            pl.BlockSpec((1, D), lambda i: (0, 0)),
        ],
        out_specs=pl.BlockSpec((RBB, D), lambda i: (i, 0)),
        out_shape=jax.ShapeDtypeStruct((N, D), jnp.float32),
    )(paggs, norm, W, b2)


def kernel(feat, edge_index, W, b):
    ei = edge_index.astype(jnp.int32)
    src_flat = ei[0]
    dst_flat = ei[1]
    ones2d = jnp.ones((CH, D), jnp.float32)
    zeros2d = jnp.zeros((CH, D), jnp.float32)
    hists = _degrees(dst_flat, ones2d, zeros2d)
    h, norm = _prenorm(hists.reshape(NC, NP, D), feat)
    paggs = _aggregate(h, src_flat, dst_flat, zeros2d)
    return _project(paggs.reshape(NC, NP, D), norm, W, b.reshape(1, D))


# speedup vs baseline: 1.1462x; 1.1440x over previous
import jax
import jax.numpy as jnp
from jax import lax
from jax.experimental import pallas as pl
from jax.experimental.pallas import tpu as pltpu
from jax.experimental.pallas import tpu_sc as plsc

N = 10000
NP = 10240
E = 320000
D = 128
NC = 2
NS = 16
L = 16
NW = NC * NS
EPW = E // NW
CH = 80
NCH = EPW // CH
RPA = NP // NS
RB = 1024

_MESH = dict(core_axis_name="c", subcore_axis_name="s")


def _deg_body(dst_hbm, ones_hbm, zeros_hbm, out_hbm, buf, iv0, iv1, hist_sh,
              semA0, semA1):
    c = lax.axis_index("c")
    s = lax.axis_index("s")
    wid = s * NC + c
    base = wid * EPW

    def start_iv(j, iv, sem):
        pltpu.make_async_copy(dst_hbm.at[pl.ds(base + j * CH, CH)], iv,
                              sem).start()

    def wait_iv(j, iv, sem):
        pltpu.make_async_copy(dst_hbm.at[pl.ds(base + j * CH, CH)], iv,
                              sem).wait()

    pltpu.sync_copy(zeros_hbm, buf)
    for q in range(RPA // CH):
        pltpu.sync_copy(buf, hist_sh.at[pl.ds(s * RPA + q * CH, CH)])
    pltpu.sync_copy(ones_hbm, buf)
    plsc.subcore_barrier()

    start_iv(0, iv0, semA0)

    def step(g, carry):
        j0 = 2 * g
        start_iv(j0 + 1, iv1, semA1)
        wait_iv(j0, iv0, semA0)
        pltpu.sync_copy(buf, hist_sh.at[iv0], add=True)
        start_iv(j0 + 2, iv0, semA0)
        wait_iv(j0 + 1, iv1, semA1)
        pltpu.sync_copy(buf, hist_sh.at[iv1], add=True)
        return carry
    lax.fori_loop(0, (NCH - 1) // 2, step, 0)
    wait_iv(NCH - 1, iv0, semA0)
    pltpu.sync_copy(buf, hist_sh.at[iv0], add=True)
    plsc.subcore_barrier()
    pltpu.sync_copy(hist_sh.at[pl.ds(s * RPA, RPA)],
                    out_hbm.at[pl.ds(c * NP + s * RPA, RPA)])


def _degrees(dst_flat, ones2d, zeros2d):
    f = pl.kernel(
        _deg_body,
        out_type=jax.ShapeDtypeStruct((NC * NP, D), jnp.float32),
        mesh=plsc.VectorSubcoreMesh(**_MESH),
        scratch_types=[
            pltpu.VMEM((CH, D), jnp.float32),
            pltpu.VMEM((CH,), jnp.int32),
            pltpu.VMEM((CH,), jnp.int32),
            pltpu.VMEM_SHARED((NP, D), jnp.float32),
            pltpu.SemaphoreType.DMA,
            pltpu.SemaphoreType.DMA,
        ],
    )
    return f(dst_flat, ones2d, zeros2d)


RBB = 1000


def _norm_body(hists_ref, feat_ref, h_ref, norm_ref):
    i = pl.program_id(0)
    blk = hists_ref[...]
    deg = blk[0, :, 0] + blk[1, :, 0]
    nrm = lax.rsqrt(jnp.maximum(deg, 1.0))
    h_ref[...] = feat_ref[...] * nrm[:, None]
    norm_ref[...] = nrm[:, None]


def _prenorm(hists, feat):
    return pl.pallas_call(
        _norm_body,
        grid=(N // RBB,),
        in_specs=[
            pl.BlockSpec((NC, RBB, D), lambda i: (0, i, 0)),
            pl.BlockSpec((RBB, D), lambda i: (i, 0)),
        ],
        out_specs=[
            pl.BlockSpec((RBB, D), lambda i: (i, 0)),
            pl.BlockSpec((RBB, 1), lambda i: (i, 0)),
        ],
        out_shape=[
            jax.ShapeDtypeStruct((N, D), jnp.float32),
            jax.ShapeDtypeStruct((N, 1), jnp.float32),
        ],
    )(hists, feat)


def _agg_body(h_hbm, src_hbm, dst_hbm, zeros_hbm, out_hbm, src_v, iv0, iv1,
              buf0, buf1, agg_sh, semA0, semA1, semG0, semG1):
    c = lax.axis_index("c")
    s = lax.axis_index("s")
    wid = s * NC + c
    base = wid * EPW

    def gather(j, buf, sem):
        return pltpu.make_async_copy(
            h_hbm.at[src_v.at[pl.ds(j * CH, CH)]], buf, sem)

    def start_iv(j, iv, sem):
        pltpu.make_async_copy(dst_hbm.at[pl.ds(base + j * CH, CH)], iv,
                              sem).start()

    def wait_iv(j, iv, sem):
        pltpu.make_async_copy(dst_hbm.at[pl.ds(base + j * CH, CH)], iv,
                              sem).wait()

    pltpu.sync_copy(zeros_hbm, buf0)
    for q in range(RPA // CH):
        pltpu.sync_copy(buf0, agg_sh.at[pl.ds(s * RPA + q * CH, CH)])
    pltpu.sync_copy(src_hbm.at[pl.ds(base, EPW)], src_v)
    plsc.subcore_barrier()

    start_iv(0, iv0, semA0)
    gather(0, buf0, semG0).start()

    def step(g, carry):
        j0 = 2 * g
        start_iv(j0 + 1, iv1, semA1)
        gather(j0 + 1, buf1, semG1).start()
        gather(j0, buf0, semG0).wait()
        wait_iv(j0, iv0, semA0)
        pltpu.sync_copy(buf0, agg_sh.at[iv0], add=True)
        start_iv(j0 + 2, iv0, semA0)
        gather(j0 + 2, buf0, semG0).start()
        gather(j0 + 1, buf1, semG1).wait()
        wait_iv(j0 + 1, iv1, semA1)
        pltpu.sync_copy(buf1, agg_sh.at[iv1], add=True)
        return carry
    lax.fori_loop(0, (NCH - 1) // 2, step, 0)
    gather(NCH - 1, buf0, semG0).wait()
    wait_iv(NCH - 1, iv0, semA0)
    pltpu.sync_copy(buf0, agg_sh.at[iv0], add=True)
    plsc.subcore_barrier()
    pltpu.sync_copy(agg_sh.at[pl.ds(s * RPA, RPA)],
                    out_hbm.at[pl.ds(c * NP + s * RPA, RPA)])


def _aggregate(h, src_flat, dst_flat, zeros2d):
    f = pl.kernel(
        _agg_body,
        out_type=jax.ShapeDtypeStruct((NC * NP, D), jnp.float32),
        mesh=plsc.VectorSubcoreMesh(**_MESH),
        scratch_types=[
            pltpu.VMEM((EPW,), jnp.int32),
            pltpu.VMEM((CH,), jnp.int32),
            pltpu.VMEM((CH,), jnp.int32),
            pltpu.VMEM((CH, D), jnp.float32),
            pltpu.VMEM((CH, D), jnp.float32),
            pltpu.VMEM_SHARED((NP, D), jnp.float32),
            pltpu.SemaphoreType.DMA,
            pltpu.SemaphoreType.DMA,
            pltpu.SemaphoreType.DMA,
            pltpu.SemaphoreType.DMA,
        ],
    )
    return f(h, src_flat, dst_flat, zeros2d)


def _out_body(paggs_ref, norm_ref, w_ref, b_ref, out_ref):
    i = pl.program_id(0)
    blk = paggs_ref[...]
    agg = blk[0] + blk[1]
    hh = agg * norm_ref[...]
    out_ref[...] = lax.dot_general(
        hh, w_ref[...], (((1,), (1,)), ((), ())),
        preferred_element_type=jnp.float32) + b_ref[...]


def _project(paggs, norm, W, b2):
    return pl.pallas_call(
        _out_body,
        grid=(N // RBB,),
        in_specs=[
            pl.BlockSpec((NC, RBB, D), lambda i: (0, i, 0)),
            pl.BlockSpec((RBB, 1), lambda i: (i, 0)),
            pl.BlockSpec((D, D), lambda i: (0, 0)),
            pl.BlockSpec((1, D), lambda i: (0, 0)),
        ],
        out_specs=pl.BlockSpec((RBB, D), lambda i: (i, 0)),
        out_shape=jax.ShapeDtypeStruct((N, D), jnp.float32),
    )(paggs, norm, W, b2)


def kernel(feat, edge_index, W, b):
    ei = edge_index.astype(jnp.int32)
    src_flat = ei[0]
    dst_flat = ei[1]
    ones2d = jnp.ones((CH, D), jnp.float32)
    zeros2d = jnp.zeros((CH, D), jnp.float32)
    hists = _degrees(dst_flat, ones2d, zeros2d)
    h, norm = _prenorm(hists.reshape(NC, NP, D), feat)
    paggs = _aggregate(h, src_flat, dst_flat, zeros2d)
    return _project(paggs.reshape(NC, NP, D), norm, W, b.reshape(1, D))
